# baseline (speedup 1.0000x reference)
import functools

import jax
import jax.numpy as jnp
from jax import lax
from jax.experimental import pallas as pl
from jax.experimental.pallas import tpu as pltpu
from jax.experimental.pallas import tpu_sc as plsc

_E = 320000
_EP = 327680
_N = 10000
_NP = 10240
_K = 5000
_KP = 5120
_CH = 20480


def _tanh_pallas(z):
    n = z.shape[0]
    npad = ((n + 1023) // 1024) * 1024
    z2 = jnp.pad(z, (0, npad - n)).reshape(npad // 128, 128)

    def body(z_ref, o_ref):
        o_ref[...] = jnp.tanh(z_ref[...])

    out = pl.pallas_call(
        body,
        out_shape=jax.ShapeDtypeStruct(z2.shape, jnp.float32),
    )(z2)
    return out.reshape(-1)[:n]


def _sc_mesh():
    return plsc.VectorSubcoreMesh(
        core_axis_name="c", subcore_axis_name="s", num_cores=1)


def _sc_gather_sorted(order1d, row1d, col1d, w1d):
    @functools.partial(
        pl.kernel,
        mesh=_sc_mesh(),
        out_type=(
            jax.ShapeDtypeStruct((_EP,), jnp.int32),
            jax.ShapeDtypeStruct((_EP,), jnp.int32),
            jax.ShapeDtypeStruct((_EP,), jnp.float32),
        ),
        scratch_types=[
            pltpu.VMEM((_CH,), jnp.int32),
            pltpu.VMEM((_CH,), jnp.int32),
            pltpu.VMEM((_CH,), jnp.float32),
            pltpu.SemaphoreType.DMA,
        ],
    )
    def k(order_hbm, row_hbm, col_hbm, w_hbm, rs_out, cs_out, ws_out,
          obuf, gbuf, gwbuf, sem):
        t = lax.axis_index("s")
        base = t * _CH
        pltpu.sync_copy(order_hbm.at[pl.ds(base, _CH)], obuf)
        pltpu.async_copy(row_hbm.at[obuf], gbuf, sem).wait()
        pltpu.sync_copy(gbuf, rs_out.at[pl.ds(base, _CH)])
        pltpu.async_copy(col_hbm.at[obuf], gbuf, sem).wait()
        pltpu.sync_copy(gbuf, cs_out.at[pl.ds(base, _CH)])
        pltpu.async_copy(w_hbm.at[obuf], gwbuf, sem).wait()
        pltpu.sync_copy(gwbuf, ws_out.at[pl.ds(base, _CH)])

    return k(order1d, row1d, col1d, w1d)


def _sc_filter(perm1d, rs1d, cs1d, ws1d, x2d, score1d, batch1d):
    @functools.partial(
        pl.kernel,
        mesh=_sc_mesh(),
        compiler_params=pltpu.CompilerParams(needs_layout_passes=False),
        out_type=(
            jax.ShapeDtypeStruct((_EP,), jnp.int32),
            jax.ShapeDtypeStruct((_EP,), jnp.int32),
            jax.ShapeDtypeStruct((_EP,), jnp.float32),
            jax.ShapeDtypeStruct((_KP, 128), jnp.float32),
            jax.ShapeDtypeStruct((_KP,), jnp.float32),
            jax.ShapeDtypeStruct((_KP,), jnp.int32),
        ),
        scratch_types=[
            pltpu.VMEM((_NP,), jnp.int32),
            pltpu.VMEM((5120,), jnp.int32),
            pltpu.VMEM((5120,), jnp.int32),
            pltpu.VMEM((5120,), jnp.float32),
            pltpu.VMEM((128,), jnp.int32),
            pltpu.VMEM((128,), jnp.int32),
            pltpu.VMEM((320,), jnp.int32),
            pltpu.VMEM((320,), jnp.float32),
            pltpu.VMEM((320,), jnp.int32),
            pltpu.VMEM((320, 128), jnp.float32),
            pltpu.VMEM_SHARED((12288,), jnp.int32),
            pltpu.SemaphoreType.DMA,
        ],
    )
    def k(perm_hbm, rs_hbm, cs_hbm, ws_hbm, x_hbm, score_hbm, batch_hbm,
          ei0_out, ei1_out, ea_out, xo_out, sp_out, bo_out,
          nid, cbuf, rbuf, wbuf, ibuf, rkbuf, pbuf, spbuf, bbuf, xrows,
          nid_sp, sem):
        t = lax.axis_index("s")
        iota16 = lax.iota(jnp.int32, 16)

        def z16(i, _):
            nid[pl.ds(i * 16, 16)] = jnp.full((16,), -1, jnp.int32)
            return 0
        lax.fori_loop(0, 48, z16, 0)
        pltpu.sync_copy(nid.at[pl.ds(0, 768)], nid_sp.at[pl.ds(t * 768, 768)])
        plsc.subcore_barrier()

        for rr in range(3):
            rowid = t + 16 * rr

            @pl.when(rowid < 40)
            def _():
                pltpu.sync_copy(perm_hbm.at[pl.ds(rowid * 128, 128)], ibuf)
                for cc in range(8):
                    pv = ibuf[pl.ds(cc * 16, 16)]
                    rk = rowid * 128 + cc * 16 + iota16
                    rkbuf[pl.ds(cc * 16, 16)] = rk
                    ibuf[pl.ds(cc * 16, 16)] = jnp.where(
                        rk < _K, pv, _NP + (rk - _K))
                pltpu.async_copy(rkbuf, nid_sp.at[ibuf], sem).wait()
        plsc.subcore_barrier()

        pltpu.sync_copy(nid_sp.at[pl.ds(0, _NP)], nid)

        for h in range(4):
            base = t * _CH + h * 5120
            pltpu.sync_copy(cs_hbm.at[pl.ds(base, 5120)], cbuf)
            pltpu.sync_copy(rs_hbm.at[pl.ds(base, 5120)], rbuf)
            pltpu.sync_copy(ws_hbm.at[pl.ds(base, 5120)], wbuf)

            def eb(v, _):
                sl = pl.ds(v * 16, 16)
                a = plsc.load_gather(nid, [cbuf[sl]])
                b = plsc.load_gather(nid, [rbuf[sl]])
                m = (a >= 0) & (b >= 0)
                cbuf[sl] = jnp.where(m, a, -1)
                rbuf[sl] = jnp.where(m, b, -1)
                wbuf[sl] = jnp.where(m, wbuf[sl], 0.0)
                return 0
            lax.fori_loop(0, 320, eb, 0)
            pltpu.sync_copy(cbuf, ei0_out.at[pl.ds(base, 5120)])
            pltpu.sync_copy(rbuf, ei1_out.at[pl.ds(base, 5120)])
            pltpu.sync_copy(wbuf, ea_out.at[pl.ds(base, 5120)])

        pltpu.sync_copy(perm_hbm.at[pl.ds(320 * t, 320)], pbuf)

        def sb(j, _):
            v = pbuf[pl.ds(j * 16, 16)]
            pbuf[pl.ds(j * 16, 16)] = jnp.where(v < _N, v, 0)
            return 0
        lax.fori_loop(0, 20, sb, 0)
        for j in range(5):
            pltpu.async_copy(
                x_hbm.at[pbuf.at[pl.ds(j * 64, 64)]],
                xrows.at[pl.ds(j * 64, 64)], sem).wait()
            pltpu.async_copy(
                score_hbm.at[pbuf.at[pl.ds(j * 64, 64)]],
                spbuf.at[pl.ds(j * 64, 64)], sem).wait()
            pltpu.async_copy(
                batch_hbm.at[pbuf.at[pl.ds(j * 64, 64)]],
                bbuf.at[pl.ds(j * 64, 64)], sem).wait()

        def mb(c, _):
            sv16 = spbuf[pl.ds(c * 16, 16)]
            for j in range(16):
                r = c * 16 + j
                for cc in range(8):
                    sl = pl.ds(cc * 16, 16)
                    xrows[r, sl] = xrows[r, sl] * sv16[j]
            return 0
        lax.fori_loop(0, 20, mb, 0)
        pltpu.sync_copy(xrows, xo_out.at[pl.ds(320 * t, 320)])
        pltpu.sync_copy(spbuf, sp_out.at[pl.ds(320 * t, 320)])
        pltpu.sync_copy(bbuf, bo_out.at[pl.ds(320 * t, 320)])

    return k(perm1d, rs1d, cs1d, ws1d, x2d, score1d, batch1d)


def kernel(x, edge_index, edge_attr, batch, p, beta):
    n = x.shape[0]
    row, col = edge_index[0], edge_index[1]
    w = edge_attr.reshape(-1)
    order = jnp.lexsort((col, row)).astype(jnp.int32)
    order_p = jnp.concatenate([order, jnp.zeros((_EP - _E,), jnp.int32)])

    rs1d, cs1d, ws1d = _sc_gather_sorted(order_p, row, col, w)
    col_s = cs1d[:_E]
    w_s = ws1d[:_E]

    score1 = (x * p).sum(axis=-1)
    score2 = jax.ops.segment_sum(w_s, col_s, num_segments=n)
    score = _tanh_pallas(beta[0] * score1 + beta[1] * score2)

    k = n // 2
    _, perm = jax.lax.top_k(score, k)
    perm_p = jnp.concatenate([perm, jnp.zeros((_KP - _K,), jnp.int32)])
    score_p = jnp.pad(score, (0, _NP - _N))
    batch_p = jnp.pad(batch, (0, _NP - _N))

    ei0, ei1, ea, xo, sp, bo = _sc_filter(
        perm_p, rs1d, cs1d, ws1d, x, score_p, batch_p)

    ei_out = jnp.stack([ei0[:_E], ei1[:_E]], axis=0)
    ea_out = ea[:_E][:, None]
    x_out = xo[:_K]
    batch_out = bo[:_K]
    return x_out, ei_out, ea_out, batch_out, perm, sp[:_K]


# speedup vs baseline: 1.0297x; 1.0297x over previous
import functools

import jax
import jax.numpy as jnp
from jax import lax
from jax.experimental import pallas as pl
from jax.experimental.pallas import tpu as pltpu
from jax.experimental.pallas import tpu_sc as plsc

_E = 320000
_EP = 327680
_N = 10000
_NP = 10240
_K = 5000
_KP = 5120
_CH = 10240


def _tanh_pallas(z):
    n = z.shape[0]
    npad = ((n + 1023) // 1024) * 1024
    z2 = jnp.pad(z, (0, npad - n)).reshape(npad // 128, 128)

    def body(z_ref, o_ref):
        o_ref[...] = jnp.tanh(z_ref[...])

    out = pl.pallas_call(
        body,
        out_shape=jax.ShapeDtypeStruct(z2.shape, jnp.float32),
    )(z2)
    return out.reshape(-1)[:n]


def _sc_mesh():
    return plsc.VectorSubcoreMesh(
        core_axis_name="c", subcore_axis_name="s", num_cores=2)


def _sc_gather_sorted(order1d, row1d, col1d, w1d):
    @functools.partial(
        pl.kernel,
        mesh=_sc_mesh(),
        out_type=(
            jax.ShapeDtypeStruct((_EP,), jnp.int32),
            jax.ShapeDtypeStruct((_EP,), jnp.int32),
            jax.ShapeDtypeStruct((_EP,), jnp.float32),
        ),
        scratch_types=[
            pltpu.VMEM((_CH,), jnp.int32),
            pltpu.VMEM((_CH,), jnp.int32),
            pltpu.VMEM((_CH,), jnp.float32),
            pltpu.SemaphoreType.DMA,
        ],
    )
    def k(order_hbm, row_hbm, col_hbm, w_hbm, rs_out, cs_out, ws_out,
          obuf, gbuf, gwbuf, sem):
        wid = lax.axis_index("s") * 2 + lax.axis_index("c")
        base = wid * _CH
        pltpu.sync_copy(order_hbm.at[pl.ds(base, _CH)], obuf)
        pltpu.async_copy(row_hbm.at[obuf], gbuf, sem).wait()
        pltpu.sync_copy(gbuf, rs_out.at[pl.ds(base, _CH)])
        pltpu.async_copy(col_hbm.at[obuf], gbuf, sem).wait()
        pltpu.sync_copy(gbuf, cs_out.at[pl.ds(base, _CH)])
        pltpu.async_copy(w_hbm.at[obuf], gwbuf, sem).wait()
        pltpu.sync_copy(gwbuf, ws_out.at[pl.ds(base, _CH)])

    return k(order1d, row1d, col1d, w1d)


def _sc_filter(perm1d, rs1d, cs1d, ws1d, x2d, score1d, batch1d):
    @functools.partial(
        pl.kernel,
        mesh=_sc_mesh(),
        compiler_params=pltpu.CompilerParams(needs_layout_passes=False),
        out_type=(
            jax.ShapeDtypeStruct((_EP,), jnp.int32),
            jax.ShapeDtypeStruct((_EP,), jnp.int32),
            jax.ShapeDtypeStruct((_EP,), jnp.float32),
            jax.ShapeDtypeStruct((_KP, 128), jnp.float32),
            jax.ShapeDtypeStruct((_KP,), jnp.float32),
            jax.ShapeDtypeStruct((_KP,), jnp.int32),
        ),
        scratch_types=[
            pltpu.VMEM((_NP,), jnp.int32),
            pltpu.VMEM((5120,), jnp.int32),
            pltpu.VMEM((5120,), jnp.int32),
            pltpu.VMEM((5120,), jnp.float32),
            pltpu.VMEM((128,), jnp.int32),
            pltpu.VMEM((128,), jnp.int32),
            pltpu.VMEM((160,), jnp.int32),
            pltpu.VMEM((160,), jnp.float32),
            pltpu.VMEM((160,), jnp.int32),
            pltpu.VMEM((160, 128), jnp.float32),
            pltpu.VMEM_SHARED((12288,), jnp.int32),
            pltpu.SemaphoreType.DMA,
        ],
    )
    def k(perm_hbm, rs_hbm, cs_hbm, ws_hbm, x_hbm, score_hbm, batch_hbm,
          ei0_out, ei1_out, ea_out, xo_out, sp_out, bo_out,
          nid, cbuf, rbuf, wbuf, ibuf, rkbuf, pbuf, spbuf, bbuf, xrows,
          nid_sp, sem):
        t = lax.axis_index("s")
        wid = t * 2 + lax.axis_index("c")
        iota16 = lax.iota(jnp.int32, 16)

        def z16(i, _):
            nid[pl.ds(i * 16, 16)] = jnp.full((16,), -1, jnp.int32)
            return 0
        lax.fori_loop(0, 48, z16, 0)
        pltpu.sync_copy(nid.at[pl.ds(0, 768)], nid_sp.at[pl.ds(t * 768, 768)])
        plsc.subcore_barrier()

        for rr in range(3):
            rowid = t + 16 * rr

            @pl.when(rowid < 40)
            def _():
                pltpu.sync_copy(perm_hbm.at[pl.ds(rowid * 128, 128)], ibuf)
                for cc in range(8):
                    pv = ibuf[pl.ds(cc * 16, 16)]
                    rk = rowid * 128 + cc * 16 + iota16
                    rkbuf[pl.ds(cc * 16, 16)] = rk
                    ibuf[pl.ds(cc * 16, 16)] = jnp.where(
                        rk < _K, pv, _NP + (rk - _K))
                pltpu.async_copy(rkbuf, nid_sp.at[ibuf], sem).wait()
        plsc.subcore_barrier()

        pltpu.sync_copy(nid_sp.at[pl.ds(0, _NP)], nid)

        for h in range(2):
            base = wid * _CH + h * 5120
            pltpu.sync_copy(cs_hbm.at[pl.ds(base, 5120)], cbuf)
            pltpu.sync_copy(rs_hbm.at[pl.ds(base, 5120)], rbuf)
            pltpu.sync_copy(ws_hbm.at[pl.ds(base, 5120)], wbuf)

            def eb(v, _):
                sl = pl.ds(v * 16, 16)
                a = plsc.load_gather(nid, [cbuf[sl]])
                b = plsc.load_gather(nid, [rbuf[sl]])
                m = (a >= 0) & (b >= 0)
                cbuf[sl] = jnp.where(m, a, -1)
                rbuf[sl] = jnp.where(m, b, -1)
                wbuf[sl] = jnp.where(m, wbuf[sl], 0.0)
                return 0
            lax.fori_loop(0, 320, eb, 0)
            pltpu.sync_copy(cbuf, ei0_out.at[pl.ds(base, 5120)])
            pltpu.sync_copy(rbuf, ei1_out.at[pl.ds(base, 5120)])
            pltpu.sync_copy(wbuf, ea_out.at[pl.ds(base, 5120)])

        pltpu.sync_copy(perm_hbm.at[pl.ds(160 * wid, 160)], pbuf)

        def sb(j, _):
            v = pbuf[pl.ds(j * 16, 16)]
            pbuf[pl.ds(j * 16, 16)] = jnp.where(v < _N, v, 0)
            return 0
        lax.fori_loop(0, 10, sb, 0)
        for j, sz in ((0, 64), (64, 64), (128, 32)):
            pltpu.async_copy(
                x_hbm.at[pbuf.at[pl.ds(j, sz)]],
                xrows.at[pl.ds(j, sz)], sem).wait()
            pltpu.async_copy(
                score_hbm.at[pbuf.at[pl.ds(j, sz)]],
                spbuf.at[pl.ds(j, sz)], sem).wait()
            pltpu.async_copy(
                batch_hbm.at[pbuf.at[pl.ds(j, sz)]],
                bbuf.at[pl.ds(j, sz)], sem).wait()

        def mb(c, _):
            sv16 = spbuf[pl.ds(c * 16, 16)]
            for j in range(16):
                r = c * 16 + j
                for cc in range(8):
                    sl = pl.ds(cc * 16, 16)
                    xrows[r, sl] = xrows[r, sl] * sv16[j]
            return 0
        lax.fori_loop(0, 10, mb, 0)
        pltpu.sync_copy(xrows, xo_out.at[pl.ds(160 * wid, 160)])
        pltpu.sync_copy(spbuf, sp_out.at[pl.ds(160 * wid, 160)])
        pltpu.sync_copy(bbuf, bo_out.at[pl.ds(160 * wid, 160)])

    return k(perm1d, rs1d, cs1d, ws1d, x2d, score1d, batch1d)


def kernel(x, edge_index, edge_attr, batch, p, beta):
    n = x.shape[0]
    row, col = edge_index[0], edge_index[1]
    w = edge_attr.reshape(-1)
    order = jnp.lexsort((col, row)).astype(jnp.int32)
    order_p = jnp.concatenate([order, jnp.zeros((_EP - _E,), jnp.int32)])

    rs1d, cs1d, ws1d = _sc_gather_sorted(order_p, row, col, w)
    col_s = cs1d[:_E]
    w_s = ws1d[:_E]

    score1 = (x * p).sum(axis=-1)
    score2 = jax.ops.segment_sum(w_s, col_s, num_segments=n)
    score = _tanh_pallas(beta[0] * score1 + beta[1] * score2)

    k = n // 2
    _, perm = jax.lax.top_k(score, k)
    perm_p = jnp.concatenate([perm, jnp.zeros((_KP - _K,), jnp.int32)])
    score_p = jnp.pad(score, (0, _NP - _N))
    batch_p = jnp.pad(batch, (0, _NP - _N))

    ei0, ei1, ea, xo, sp, bo = _sc_filter(
        perm_p, rs1d, cs1d, ws1d, x, score_p, batch_p)

    ei_out = jnp.stack([ei0[:_E], ei1[:_E]], axis=0)
    ea_out = ea[:_E][:, None]
    x_out = xo[:_K]
    batch_out = bo[:_K]
    return x_out, ei_out, ea_out, batch_out, perm, sp[:_K]
